# baseline (speedup 1.0000x reference)
import jax
import jax.numpy as jnp
from jax.experimental import pallas as pl
from jax.experimental.pallas import tpu as pltpu

_TB = 32


def _vq_onehot_body(x_ref, cb_ref, out_ref, c2_ref, cbm2_ref):
    K = cb_ref.shape[0]
    i = pl.program_id(0)

    @pl.when(i == 0)
    def _():
        cb = cb_ref[...]
        c2_ref[...] = jnp.sum(cb * cb, axis=-1)[None, :]
        cbm2_ref[...] = cb * (-2.0)

    B, TB, D = x_ref.shape
    cbm2 = cbm2_ref[...]
    c2 = c2_ref[...]
    lane = None
    rows = []
    for b in range(B):
        x = x_ref[b]
        z2 = jnp.sum(x * x, axis=-1, keepdims=True)
        crossm2 = jax.lax.dot_general(
            x, cbm2, (((1,), (1,)), ((), ())),
            preferred_element_type=jnp.float32)
        dist = z2 + crossm2 + c2
        minval = jnp.min(dist, axis=-1, keepdims=True)
        if lane is None:
            lane = jax.lax.broadcasted_iota(jnp.int32, dist.shape, 1)
        codes_b = jnp.min(jnp.where(dist == minval, lane, K),
                          axis=-1, keepdims=True)
        rows.append(codes_b.reshape(1, TB))
    codes = jnp.concatenate(rows, axis=0)
    kiota = jax.lax.broadcasted_iota(jnp.int32, (B, K), 1)
    for t in range(_TB):
        target = codes[:, t:t + 1]
        out_ref[:, t * K:(t + 1) * K] = (target == kiota).astype(out_ref.dtype)


def kernel(input, codebook):
    B, T, D = input.shape
    K = codebook.shape[0]
    onehot = pl.pallas_call(
        _vq_onehot_body,
        grid=(T // _TB,),
        in_specs=[
            pl.BlockSpec((B, _TB, D), lambda i: (0, i, 0)),
            pl.BlockSpec((K, D), lambda i: (0, 0)),
        ],
        out_specs=pl.BlockSpec((B, _TB * K), lambda i: (0, i)),
        out_shape=jax.ShapeDtypeStruct((B, T * K), jnp.int32),
        scratch_shapes=[pltpu.VMEM((1, K), jnp.float32),
                        pltpu.VMEM((K, D), jnp.float32)],
    )(input, codebook)
    return onehot.astype(jnp.int64)


# speedup vs baseline: 1.5909x; 1.5909x over previous
import jax
import jax.numpy as jnp
from jax.experimental import pallas as pl
from jax.experimental.pallas import tpu as pltpu

_TB = 32


def _vq_onehot_body(x_ref, cb_ref, out_ref, c2_ref, cbm2_ref, perm_ref):
    K = cb_ref.shape[0]
    i = pl.program_id(0)

    @pl.when(i == 0)
    def _():
        cb = cb_ref[...]
        c2_ref[...] = jnp.sum(cb * cb, axis=-1)[None, :]
        cbm2_ref[...] = cb * (-2.0)

    B, TB, D = x_ref.shape
    R = B * TB

    @pl.when(i == 0)
    def _():
        r = jax.lax.broadcasted_iota(jnp.int32, (R, R), 0)
        j = jax.lax.broadcasted_iota(jnp.int32, (R, R), 1)
        src = (r % B) * TB + (r // B)
        perm_ref[...] = (j == src).astype(jnp.float32)

    xf = x_ref[...].reshape(R, D)
    x = jax.lax.dot_general(
        perm_ref[...], xf, (((1,), (0,)), ((), ())),
        preferred_element_type=jnp.float32)
    z2 = jnp.sum(x * x, axis=-1, keepdims=True)
    crossm2 = jax.lax.dot_general(
        x, cbm2_ref[...], (((1,), (1,)), ((), ())),
        preferred_element_type=jnp.float32)
    dist = z2 + crossm2 + c2_ref[...]
    minval = jnp.min(dist, axis=-1, keepdims=True)
    lane = jax.lax.broadcasted_iota(jnp.int32, dist.shape, 1)
    codes = jnp.min(jnp.where(dist == minval, lane, K),
                    axis=-1, keepdims=True)
    kiota = jax.lax.broadcasted_iota(jnp.int32, (B, K), 1)
    for t in range(_TB):
        target = codes[t * B:(t + 1) * B]
        out_ref[:, t * K:(t + 1) * K] = (target == kiota).astype(out_ref.dtype)


def kernel(input, codebook):
    B, T, D = input.shape
    K = codebook.shape[0]
    onehot = pl.pallas_call(
        _vq_onehot_body,
        grid=(T // _TB,),
        in_specs=[
            pl.BlockSpec((B, _TB, D), lambda i: (0, i, 0)),
            pl.BlockSpec((K, D), lambda i: (0, 0)),
        ],
        out_specs=pl.BlockSpec((B, _TB * K), lambda i: (0, i)),
        out_shape=jax.ShapeDtypeStruct((B, T * K), jnp.int32),
        scratch_shapes=[pltpu.VMEM((1, K), jnp.float32),
                        pltpu.VMEM((K, D), jnp.float32),
                        pltpu.VMEM((B * _TB, B * _TB), jnp.float32)],
    )(input, codebook)
    return onehot.astype(jnp.int64)
